# Initial kernel scaffold; baseline (speedup 1.0000x reference)
import jax
import jax.numpy as jnp
from jax.experimental import pallas as pl


def kernel(inputs, pos_table):
    raise NotImplementedError("write your pallas kernel here")


# speedup vs baseline: 1.4534x; 1.4534x over previous
import jax
import jax.numpy as jnp
from jax.experimental import pallas as pl
from jax.experimental.pallas import tpu as pltpu

_BLOCK_S = 512


def _add_kernel(in_ref, pos_ref, out_ref):
    out_ref[...] = in_ref[...] + pos_ref[...]


def kernel(inputs, pos_table):
    batch, seq, emb = inputs.shape
    bs = _BLOCK_S
    grid = (seq // bs, batch)
    return pl.pallas_call(
        _add_kernel,
        grid=grid,
        in_specs=[
            pl.BlockSpec((1, bs, emb), lambda s, b: (b, s, 0)),
            pl.BlockSpec((bs, emb), lambda s, b: (s, 0)),
        ],
        out_specs=pl.BlockSpec((1, bs, emb), lambda s, b: (b, s, 0)),
        out_shape=jax.ShapeDtypeStruct(inputs.shape, inputs.dtype),
        compiler_params=pltpu.CompilerParams(
            dimension_semantics=("arbitrary", "arbitrary"),
        ),
    )(inputs, pos_table)


# speedup vs baseline: 1.6752x; 1.1526x over previous
import jax
import jax.numpy as jnp
from jax.experimental import pallas as pl
from jax.experimental.pallas import tpu as pltpu

_BLOCK_S = 1024


def _add_kernel(in_ref, pos_ref, out_ref):
    out_ref[...] = in_ref[...] + pos_ref[...]


def kernel(inputs, pos_table):
    batch, seq, emb = inputs.shape
    bs = _BLOCK_S
    grid = (seq // bs, batch)
    return pl.pallas_call(
        _add_kernel,
        grid=grid,
        in_specs=[
            pl.BlockSpec((1, bs, emb), lambda s, b: (b, s, 0)),
            pl.BlockSpec((bs, emb), lambda s, b: (s, 0)),
        ],
        out_specs=pl.BlockSpec((1, bs, emb), lambda s, b: (b, s, 0)),
        out_shape=jax.ShapeDtypeStruct(inputs.shape, inputs.dtype),
        compiler_params=pltpu.CompilerParams(
            dimension_semantics=("arbitrary", "arbitrary"),
        ),
    )(inputs, pos_table)


# speedup vs baseline: 1.7942x; 1.0710x over previous
import jax
import jax.numpy as jnp
from jax.experimental import pallas as pl
from jax.experimental.pallas import tpu as pltpu

_BLOCK_S = 2048


def _add_kernel(in_ref, pos_ref, out_ref):
    out_ref[...] = in_ref[...] + pos_ref[...]


def kernel(inputs, pos_table):
    batch, seq, emb = inputs.shape
    bs = _BLOCK_S
    grid = (seq // bs, batch)
    return pl.pallas_call(
        _add_kernel,
        grid=grid,
        in_specs=[
            pl.BlockSpec((1, bs, emb), lambda s, b: (b, s, 0)),
            pl.BlockSpec((bs, emb), lambda s, b: (s, 0)),
        ],
        out_specs=pl.BlockSpec((1, bs, emb), lambda s, b: (b, s, 0)),
        out_shape=jax.ShapeDtypeStruct(inputs.shape, inputs.dtype),
        compiler_params=pltpu.CompilerParams(
            dimension_semantics=("arbitrary", "arbitrary"),
        ),
    )(inputs, pos_table)


# speedup vs baseline: 1.8024x; 1.0046x over previous
import jax
import jax.numpy as jnp
from jax.experimental import pallas as pl
from jax.experimental.pallas import tpu as pltpu

_BLOCK_S = 1024


def _add_kernel(in_ref, pos_ref, out_ref):
    out_ref[...] = in_ref[...] + pos_ref[...][None, :, :]


def kernel(inputs, pos_table):
    batch, seq, emb = inputs.shape
    bs = _BLOCK_S
    grid = (seq // bs,)
    return pl.pallas_call(
        _add_kernel,
        grid=grid,
        in_specs=[
            pl.BlockSpec((batch, bs, emb), lambda s: (0, s, 0)),
            pl.BlockSpec((bs, emb), lambda s: (s, 0)),
        ],
        out_specs=pl.BlockSpec((batch, bs, emb), lambda s: (0, s, 0)),
        out_shape=jax.ShapeDtypeStruct(inputs.shape, inputs.dtype),
        compiler_params=pltpu.CompilerParams(
            dimension_semantics=("arbitrary",),
        ),
    )(inputs, pos_table)


# speedup vs baseline: 1.8031x; 1.0004x over previous
import jax
import jax.numpy as jnp
from jax.experimental import pallas as pl
from jax.experimental.pallas import tpu as pltpu

_BLOCK_S = 512


def _add_kernel(in_ref, pos_ref, out_ref):
    out_ref[...] = in_ref[...] + pos_ref[...][None, :, :]


def kernel(inputs, pos_table):
    batch, seq, emb = inputs.shape
    bs = _BLOCK_S
    grid = (seq // bs,)
    return pl.pallas_call(
        _add_kernel,
        grid=grid,
        in_specs=[
            pl.BlockSpec((batch, bs, emb), lambda s: (0, s, 0)),
            pl.BlockSpec((bs, emb), lambda s: (s, 0)),
        ],
        out_specs=pl.BlockSpec((batch, bs, emb), lambda s: (0, s, 0)),
        out_shape=jax.ShapeDtypeStruct(inputs.shape, inputs.dtype),
        compiler_params=pltpu.CompilerParams(
            dimension_semantics=("arbitrary",),
        ),
    )(inputs, pos_table)
